# initial kernel scaffold (unmeasured)
import jax
import jax.numpy as jnp
from jax import lax
from jax.experimental import pallas as pl
from jax.experimental.pallas import tpu as pltpu

N_DEV = 8
SQ = 1024
SKV = 1024
DH = 128
HG = 8
DM = 1024
WIN = 128
SCALE = 0.08838834764831843

QB = 256
KW = 512
KSTART = (0, 128, 384, 512)


def _band_masks():
    masks = []
    for qb in range(4):
        r = lax.broadcasted_iota(jnp.int32, (QB, KW), 0)
        c = lax.broadcasted_iota(jnp.int32, (QB, KW), 1)
        qi = qb * QB + r
        ki = KSTART[qb] + c
        masks.append(jnp.abs(qi - ki) <= WIN)
    return masks


def _body(x_ref, wq_ref, wo_ref, kt_hbm, vt_hbm, out_ref,
          wq_ring, wo_ring, kbuf, vbuf, ctx_ref,
          wq_ssem, wq_rsem, wo_ssem, wo_rsem, ksem, vsem):
    i = lax.axis_index("i")
    right = lax.rem(i + 1, N_DEV)
    left = lax.rem(i + N_DEV - 1, N_DEV)

    barrier = pltpu.get_barrier_semaphore()
    for nbr in (left, right):
        pl.semaphore_signal(barrier, inc=1, device_id=(nbr,),
                            device_id_type=pl.DeviceIdType.MESH)
    pl.semaphore_wait(barrier, 2)

    offs = (0, 4, 3, -3, 2, -2, 1, -1)

    def kv_copies(order_idx):
        g = lax.rem(i + offs[order_idx] + N_DEV, N_DEV)
        slot = order_idx % 2
        kc = pltpu.make_async_copy(kt_hbm.at[pl.ds(g * HG, HG)],
                                   kbuf.at[pl.ds(slot * HG, HG)],
                                   ksem.at[slot])
        vc = pltpu.make_async_copy(vt_hbm.at[pl.ds(g * HG, HG)],
                                   vbuf.at[pl.ds(slot * HG, HG)],
                                   vsem.at[slot])
        return kc, vc

    def kv_start(order_idx):
        kc, vc = kv_copies(order_idx)
        kc.start()
        vc.start()

    masks = _band_masks()

    def compute_group(order_idx, wq_mat, wo_mat, first):
        slot = order_idx % 2
        kc, vc = kv_copies(order_idx)
        kc.wait()
        vc.wait()
        q = jnp.dot(x_ref[...], wq_mat,
                    preferred_element_type=jnp.float32).astype(jnp.bfloat16)

        def head(h, carry):
            idx = slot * HG + h
            khd = kbuf[idx]
            vhd = vbuf[idx]
            qcol = h * DH
            for qb in range(4):
                qblk = lax.dynamic_slice(q, (qb * QB, qcol), (QB, DH))
                kwin = khd[KSTART[qb]:KSTART[qb] + KW, :]
                s = lax.dot_general(qblk, kwin, (((1,), (1,)), ((), ())),
                                    preferred_element_type=jnp.float32)
                s = jnp.where(masks[qb], s * SCALE, -1e9)
                m = jnp.max(s, axis=1, keepdims=True)
                e = jnp.exp(s - m)
                den = jnp.sum(e, axis=1, keepdims=True)
                w = (e / den).astype(jnp.bfloat16)
                vwin = vhd[KSTART[qb]:KSTART[qb] + KW, :]
                c = jnp.dot(w, vwin, preferred_element_type=jnp.float32)
                pl.store(ctx_ref, (pl.ds(qb * QB, QB), pl.ds(qcol, DH)),
                         c.astype(jnp.bfloat16))
            return carry

        lax.fori_loop(0, HG, head, 0)
        if order_idx + 2 < N_DEV:
            kv_start(order_idx + 2)
        partial = jnp.dot(ctx_ref[...], wo_mat,
                          preferred_element_type=jnp.float32)
        if first:
            out_ref[0] = partial
        else:
            out_ref[0] = out_ref[0] + partial

    def fwd(ring, ssem, rsem, src_ref, dst_slot, tgt):
        d = pltpu.make_async_remote_copy(
            src_ref=src_ref, dst_ref=ring.at[dst_slot],
            send_sem=ssem.at[dst_slot], recv_sem=rsem.at[dst_slot],
            device_id=(tgt,), device_id_type=pl.DeviceIdType.MESH)
        d.start()
        return d

    def recv_wait(ring, ssem, rsem, slot):
        pltpu.make_async_remote_copy(
            src_ref=ring.at[slot], dst_ref=ring.at[slot],
            send_sem=ssem.at[slot], recv_sem=rsem.at[slot],
            device_id=(right,), device_id_type=pl.DeviceIdType.MESH,
        ).wait_recv()

    kv_start(0)
    kv_start(1)

    sends = []
    sends.append(fwd(wq_ring, wq_ssem, wq_rsem, wq_ref, 0, right))
    sends.append(fwd(wo_ring, wo_ssem, wo_rsem, wo_ref, 0, left))

    compute_group(0, wq_ref[...], wo_ref[...], True)

    for t in range(1, N_DEV):
        recv_wait(wq_ring, wq_ssem, wq_rsem, t - 1)
        recv_wait(wo_ring, wo_ssem, wo_rsem, t - 1)
        if t <= N_DEV - 2:
            sends.append(fwd(wq_ring, wq_ssem, wq_rsem,
                             wq_ring.at[t - 1], t, right))
            sends.append(fwd(wo_ring, wo_ssem, wo_rsem,
                             wo_ring.at[t - 1], t, left))
        if t == 4:
            compute_group(1, wq_ring[3], wo_ring[3], False)
        elif t >= 5:
            ia = 2 * (t - 4)
            compute_group(ia, wq_ring[t - 1], wo_ring[7 - t], False)
            compute_group(ia + 1, wq_ring[7 - t], wo_ring[t - 1], False)

    for d in sends:
        d.wait_send()


def kernel(x, Wq, K_ext, V_ext, Wo):
    i = lax.axis_index("i")
    xb = x[0].astype(jnp.bfloat16)
    wq = Wq.astype(jnp.bfloat16)
    wo = Wo.astype(jnp.bfloat16)
    kb = lax.dynamic_index_in_dim(K_ext, i, 0, keepdims=False)
    vb = lax.dynamic_index_in_dim(V_ext, i, 0, keepdims=False)
    kt = jnp.transpose(kb, (1, 0, 2)).astype(jnp.bfloat16)
    vt = jnp.transpose(vb, (1, 0, 2)).astype(jnp.bfloat16)

    return pl.pallas_call(
        _body,
        out_shape=jax.ShapeDtypeStruct((1, SQ, DM), jnp.float32),
        in_specs=[
            pl.BlockSpec(memory_space=pltpu.VMEM),
            pl.BlockSpec(memory_space=pltpu.VMEM),
            pl.BlockSpec(memory_space=pltpu.VMEM),
            pl.BlockSpec(memory_space=pltpu.ANY),
            pl.BlockSpec(memory_space=pltpu.ANY),
        ],
        out_specs=pl.BlockSpec(memory_space=pltpu.VMEM),
        scratch_shapes=[
            pltpu.VMEM((N_DEV - 1, DM, DM), jnp.bfloat16),
            pltpu.VMEM((N_DEV - 1, DM, DM), jnp.bfloat16),
            pltpu.VMEM((2 * HG, SKV, DH), jnp.bfloat16),
            pltpu.VMEM((2 * HG, SKV, DH), jnp.bfloat16),
            pltpu.VMEM((SQ, DM), jnp.bfloat16),
            pltpu.SemaphoreType.DMA((N_DEV - 1,)),
            pltpu.SemaphoreType.DMA((N_DEV - 1,)),
            pltpu.SemaphoreType.DMA((N_DEV - 1,)),
            pltpu.SemaphoreType.DMA((N_DEV - 1,)),
            pltpu.SemaphoreType.DMA((2,)),
            pltpu.SemaphoreType.DMA((2,)),
        ],
        compiler_params=pltpu.CompilerParams(collective_id=0),
    )(xb, wq, wo, kt, vt)


# baseline (device time: 350790 ns/iter reference)
import jax
import jax.numpy as jnp
from jax import lax
from jax.experimental import pallas as pl
from jax.experimental.pallas import tpu as pltpu

N_DEV = 8
SQ = 1024
SKV = 1024
DH = 128
HG = 8
DM = 1024
WIN = 128
SCALE = 0.08838834764831843

QB = 256
KW = 512
KSTART = (0, 128, 384, 512)


def _band_mask(qb):
    r = lax.broadcasted_iota(jnp.int32, (QB, KW), 0)
    c = lax.broadcasted_iota(jnp.int32, (QB, KW), 1)
    qi = qb * QB + r
    ki = KSTART[qb] + c
    return jnp.abs(qi - ki) <= WIN


def _body(x_ref, wq_ref, wo_ref, kt_hbm, vt_hbm, out_ref,
          wq_ring, wo_ring, kbuf, vbuf, ctx_ref, q_ref,
          wq_ssem, wq_rsem, wo_ssem, wo_rsem, ksem, vsem):
    i = lax.axis_index("i")
    right = lax.rem(i + 1, N_DEV)
    left = lax.rem(i + N_DEV - 1, N_DEV)

    barrier = pltpu.get_barrier_semaphore()
    for nbr in (left, right):
        pl.semaphore_signal(barrier, inc=1, device_id=(nbr,),
                            device_id_type=pl.DeviceIdType.MESH)
    pl.semaphore_wait(barrier, 2)

    offs = (0, 4, 3, -3, 2, -2, 1, -1)

    def kv_copies(order_idx):
        g = lax.rem(i + offs[order_idx] + N_DEV, N_DEV)
        slot = order_idx % 2
        kc = pltpu.make_async_copy(kt_hbm.at[pl.ds(g * HG, HG)],
                                   kbuf.at[pl.ds(slot * HG, HG)],
                                   ksem.at[slot])
        vc = pltpu.make_async_copy(vt_hbm.at[pl.ds(g * HG, HG)],
                                   vbuf.at[pl.ds(slot * HG, HG)],
                                   vsem.at[slot])
        return kc, vc

    def kv_start(order_idx):
        kc, vc = kv_copies(order_idx)
        kc.start()
        vc.start()

    def compute_group(order_idx, wq_mat, wo_mat, first):
        slot = order_idx % 2
        kc, vc = kv_copies(order_idx)
        kc.wait()
        vc.wait()
        for qb in range(4):
            rows = pl.ds(qb * QB, QB)
            q_ref[rows, :] = jnp.dot(x_ref[rows, :], wq_mat,
                                     preferred_element_type=jnp.float32
                                     ).astype(jnp.bfloat16)

        def head(h, carry):
            idx = slot * HG + h
            khd = kbuf[idx]
            vhd = vbuf[idx]
            qcol = h * DH
            for qb in range(4):
                qblk = q_ref[pl.ds(qb * QB, QB), pl.ds(qcol, DH)]
                kwin = khd[KSTART[qb]:KSTART[qb] + KW, :]
                s = lax.dot_general(qblk, kwin, (((1,), (1,)), ((), ())),
                                    preferred_element_type=jnp.float32)
                s = jnp.where(_band_mask(qb), s * SCALE, -1e9)
                m = jnp.max(s, axis=1, keepdims=True)
                e = jnp.exp(s - m)
                den = jnp.sum(e, axis=1, keepdims=True)
                w = (e / den).astype(jnp.bfloat16)
                vwin = vhd[KSTART[qb]:KSTART[qb] + KW, :]
                c = jnp.dot(w, vwin, preferred_element_type=jnp.float32)
                ctx_ref[pl.ds(qb * QB, QB), pl.ds(qcol, DH)] = (
                    c.astype(jnp.bfloat16))
            return carry

        lax.fori_loop(0, HG, head, 0)
        if order_idx + 2 < N_DEV:
            kv_start(order_idx + 2)
        for qb in range(4):
            rows = pl.ds(qb * QB, QB)
            partial = jnp.dot(ctx_ref[rows, :], wo_mat,
                              preferred_element_type=jnp.float32)
            if first:
                out_ref[0, rows, :] = partial
            else:
                out_ref[0, rows, :] = out_ref[0, rows, :] + partial

    def fwd(ring, ssem, rsem, src_ref, dst_slot, tgt):
        d = pltpu.make_async_remote_copy(
            src_ref=src_ref, dst_ref=ring.at[dst_slot],
            send_sem=ssem.at[dst_slot], recv_sem=rsem.at[dst_slot],
            device_id=(tgt,), device_id_type=pl.DeviceIdType.MESH)
        d.start()
        return d

    def recv_wait(ring, ssem, rsem, slot):
        pltpu.make_async_remote_copy(
            src_ref=ring.at[slot], dst_ref=ring.at[slot],
            send_sem=ssem.at[slot], recv_sem=rsem.at[slot],
            device_id=(right,), device_id_type=pl.DeviceIdType.MESH,
        ).wait_recv()

    kv_start(0)
    kv_start(1)

    sends = []
    sends.append(fwd(wq_ring, wq_ssem, wq_rsem, wq_ref, 0, right))
    sends.append(fwd(wo_ring, wo_ssem, wo_rsem, wo_ref, 0, left))

    compute_group(0, wq_ref[...], wo_ref[...], True)

    for t in range(1, N_DEV):
        recv_wait(wq_ring, wq_ssem, wq_rsem, t - 1)
        recv_wait(wo_ring, wo_ssem, wo_rsem, t - 1)
        if t <= N_DEV - 2:
            sends.append(fwd(wq_ring, wq_ssem, wq_rsem,
                             wq_ring.at[t - 1], t, right))
            sends.append(fwd(wo_ring, wo_ssem, wo_rsem,
                             wo_ring.at[t - 1], t, left))
        if t == 4:
            compute_group(1, wq_ring[3], wo_ring[3], False)
        elif t >= 5:
            ia = 2 * (t - 4)
            compute_group(ia, wq_ring[t - 1], wo_ring[7 - t], False)
            compute_group(ia + 1, wq_ring[7 - t], wo_ring[t - 1], False)

    for d in sends:
        d.wait_send()


def kernel(x, Wq, K_ext, V_ext, Wo):
    i = lax.axis_index("i")
    xb = x[0].astype(jnp.bfloat16)
    wq = Wq.astype(jnp.bfloat16)
    wo = Wo.astype(jnp.bfloat16)
    kb = lax.dynamic_index_in_dim(K_ext, i, 0, keepdims=False)
    vb = lax.dynamic_index_in_dim(V_ext, i, 0, keepdims=False)
    kt = jnp.transpose(kb, (1, 0, 2)).astype(jnp.bfloat16)
    vt = jnp.transpose(vb, (1, 0, 2)).astype(jnp.bfloat16)

    return pl.pallas_call(
        _body,
        out_shape=jax.ShapeDtypeStruct((1, SQ, DM), jnp.float32),
        in_specs=[
            pl.BlockSpec(memory_space=pltpu.VMEM),
            pl.BlockSpec(memory_space=pltpu.VMEM),
            pl.BlockSpec(memory_space=pltpu.VMEM),
            pl.BlockSpec(memory_space=pl.ANY),
            pl.BlockSpec(memory_space=pl.ANY),
        ],
        out_specs=pl.BlockSpec(memory_space=pltpu.VMEM),
        scratch_shapes=[
            pltpu.VMEM((N_DEV - 1, DM, DM), jnp.bfloat16),
            pltpu.VMEM((N_DEV - 1, DM, DM), jnp.bfloat16),
            pltpu.VMEM((2 * HG, SKV, DH), jnp.bfloat16),
            pltpu.VMEM((2 * HG, SKV, DH), jnp.bfloat16),
            pltpu.VMEM((SQ, DM), jnp.bfloat16),
            pltpu.VMEM((SQ, DM), jnp.bfloat16),
            pltpu.SemaphoreType.DMA((N_DEV - 1,)),
            pltpu.SemaphoreType.DMA((N_DEV - 1,)),
            pltpu.SemaphoreType.DMA((N_DEV - 1,)),
            pltpu.SemaphoreType.DMA((N_DEV - 1,)),
            pltpu.SemaphoreType.DMA((2,)),
            pltpu.SemaphoreType.DMA((2,)),
        ],
        compiler_params=pltpu.CompilerParams(
            collective_id=0,
            vmem_limit_bytes=60 * 1024 * 1024,
        ),
    )(xb, wq, wo, kt, vt)


# device time: 292000 ns/iter; 1.2013x vs baseline; 1.2013x over previous
import jax
import jax.numpy as jnp
from jax import lax
from jax.experimental import pallas as pl
from jax.experimental.pallas import tpu as pltpu

N_DEV = 8
SQ = 1024
SKV = 1024
DH = 128
HG = 8
DM = 1024
WIN = 128
SCALE = 0.08838834764831843

QB = 256
KW = 512
KSTART = (0, 128, 384, 512)


def _band_mask(qb):
    r = lax.broadcasted_iota(jnp.int32, (QB, KW), 0)
    c = lax.broadcasted_iota(jnp.int32, (QB, KW), 1)
    qi = qb * QB + r
    ki = KSTART[qb] + c
    return jnp.abs(qi - ki) <= WIN


def _body(x_ref, wq_ref, wo_ref, kt_hbm, vt_hbm, out_ref,
          wq_ring, wo_ring, kbuf, vbuf, ctx_ref, q_ref, bias_ref, stash,
          wq_ssem, wq_rsem, wo_ssem, wo_rsem, ksem, vsem):
    i = lax.axis_index("i")
    right = lax.rem(i + 1, N_DEV)
    left = lax.rem(i + N_DEV - 1, N_DEV)

    for qb in range(4):
        bias_ref[qb] = jnp.where(_band_mask(qb), 0.0, -1e9
                                 ).astype(jnp.bfloat16)

    barrier = pltpu.get_barrier_semaphore()
    for nbr in (left, right):
        pl.semaphore_signal(barrier, inc=1, device_id=(nbr,),
                            device_id_type=pl.DeviceIdType.MESH)
    pl.semaphore_wait(barrier, 2)

    offs = (0, -1, -2, 4, 3, -3, 2, 1)

    def kv_copies(order_idx):
        g = lax.rem(i + offs[order_idx] + N_DEV, N_DEV)
        kc = pltpu.make_async_copy(kt_hbm.at[:, pl.ds(g * DM, DM)],
                                   kbuf, ksem)
        vc = pltpu.make_async_copy(vt_hbm.at[:, pl.ds(g * DM, DM)],
                                   vbuf, vsem)
        return kc, vc

    def kv_start(order_idx):
        kc, vc = kv_copies(order_idx)
        kc.start()
        vc.start()

    def ctx_part(order_idx, wq_mat, dst):
        kc, vc = kv_copies(order_idx)
        kc.wait()
        vc.wait()
        for qb in range(4):
            rows = pl.ds(qb * QB, QB)
            q_ref[rows, :] = (jnp.dot(x_ref[rows, :], wq_mat,
                                      preferred_element_type=jnp.float32)
                              * SCALE).astype(jnp.bfloat16)

        def head(h, carry):
            qcol = h * DH
            for qb in range(4):
                qblk = q_ref[pl.ds(qb * QB, QB), pl.ds(qcol, DH)]
                kwin = kbuf[pl.ds(KSTART[qb], KW), pl.ds(qcol, DH)]
                s = lax.dot_general(qblk, kwin, (((1,), (1,)), ((), ())),
                                    preferred_element_type=jnp.float32)
                e = jnp.exp(s + bias_ref[qb])
                den = jnp.sum(e, axis=1, keepdims=True)
                w = (e / den).astype(jnp.bfloat16)
                vwin = vbuf[pl.ds(KSTART[qb], KW), pl.ds(qcol, DH)]
                c = jnp.dot(w, vwin, preferred_element_type=jnp.float32)
                dst[pl.ds(qb * QB, QB), pl.ds(qcol, DH)] = (
                    c.astype(jnp.bfloat16))
            return carry

        lax.fori_loop(0, HG, head, 0)
        if order_idx + 1 < N_DEV:
            kv_start(order_idx + 1)

    def proj_part(src, wo_mat, first=False):
        for qb in range(4):
            rows = pl.ds(qb * QB, QB)
            partial = jnp.dot(src[rows, :], wo_mat,
                              preferred_element_type=jnp.float32)
            if first:
                out_ref[0, rows, :] = partial
            else:
                out_ref[0, rows, :] = out_ref[0, rows, :] + partial

    def fwd(ring, ssem, rsem, src_ref, dst_slot, tgt):
        d = pltpu.make_async_remote_copy(
            src_ref=src_ref, dst_ref=ring.at[dst_slot],
            send_sem=ssem.at[dst_slot], recv_sem=rsem.at[dst_slot],
            device_id=(tgt,), device_id_type=pl.DeviceIdType.MESH)
        d.start()
        return d

    def recv_wait(ring, ssem, rsem, slot):
        pltpu.make_async_remote_copy(
            src_ref=ring.at[slot], dst_ref=ring.at[slot],
            send_sem=ssem.at[slot], recv_sem=rsem.at[slot],
            device_id=(right,), device_id_type=pl.DeviceIdType.MESH,
        ).wait_recv()

    kv_start(0)

    sends = []
    sends.append(fwd(wq_ring, wq_ssem, wq_rsem, wq_ref, 0, right))
    sends.append(fwd(wo_ring, wo_ssem, wo_rsem, wo_ref, 0, left))

    ctx_part(0, wq_ref[...], ctx_ref)
    proj_part(ctx_ref, wo_ref[...], first=True)

    for t in range(1, N_DEV):
        recv_wait(wq_ring, wq_ssem, wq_rsem, t - 1)
        recv_wait(wo_ring, wo_ssem, wo_rsem, t - 1)
        if t <= N_DEV - 2:
            sends.append(fwd(wq_ring, wq_ssem, wq_rsem,
                             wq_ring.at[t - 1], t, right))
            sends.append(fwd(wo_ring, wo_ssem, wo_rsem,
                             wo_ring.at[t - 1], t, left))
        if t <= 2:
            ctx_part(t, wq_ring[t - 1], stash.at[t - 1])
        elif t == 4:
            ctx_part(3, wq_ring[3], ctx_ref)
            proj_part(ctx_ref, wo_ring[3])
        elif t == 5:
            ctx_part(4, wq_ring[4], ctx_ref)
            proj_part(ctx_ref, wo_ring[2])
            ctx_part(5, wq_ring[2], ctx_ref)
            proj_part(ctx_ref, wo_ring[4])
        elif t >= 6:
            proj_part(stash.at[7 - t], wo_ring[t - 1])
            ctx_part(t, wq_ring[t - 1], ctx_ref)
            proj_part(ctx_ref, wo_ring[7 - t])

    for d in sends:
        d.wait_send()


def kernel(x, Wq, K_ext, V_ext, Wo):
    i = lax.axis_index("i")
    xb = x[0].astype(jnp.bfloat16)
    wq = Wq.astype(jnp.bfloat16)
    wo = Wo.astype(jnp.bfloat16)
    kb = lax.dynamic_index_in_dim(K_ext, i, 0, keepdims=False)
    vb = lax.dynamic_index_in_dim(V_ext, i, 0, keepdims=False)
    kt = kb.reshape(SKV, 64 * DH).astype(jnp.bfloat16)
    vt = vb.reshape(SKV, 64 * DH).astype(jnp.bfloat16)

    return pl.pallas_call(
        _body,
        out_shape=jax.ShapeDtypeStruct((1, SQ, DM), jnp.float32),
        in_specs=[
            pl.BlockSpec(memory_space=pltpu.VMEM),
            pl.BlockSpec(memory_space=pltpu.VMEM),
            pl.BlockSpec(memory_space=pltpu.VMEM),
            pl.BlockSpec(memory_space=pl.ANY),
            pl.BlockSpec(memory_space=pl.ANY),
        ],
        out_specs=pl.BlockSpec(memory_space=pltpu.VMEM),
        scratch_shapes=[
            pltpu.VMEM((N_DEV - 1, DM, DM), jnp.bfloat16),
            pltpu.VMEM((N_DEV - 1, DM, DM), jnp.bfloat16),
            pltpu.VMEM((SKV, DM), jnp.bfloat16),
            pltpu.VMEM((SKV, DM), jnp.bfloat16),
            pltpu.VMEM((SQ, DM), jnp.bfloat16),
            pltpu.VMEM((SQ, DM), jnp.bfloat16),
            pltpu.VMEM((4, QB, KW), jnp.bfloat16),
            pltpu.VMEM((2, SQ, DM), jnp.bfloat16),
            pltpu.SemaphoreType.DMA((N_DEV - 1,)),
            pltpu.SemaphoreType.DMA((N_DEV - 1,)),
            pltpu.SemaphoreType.DMA((N_DEV - 1,)),
            pltpu.SemaphoreType.DMA((N_DEV - 1,)),
            pltpu.SemaphoreType.DMA,
            pltpu.SemaphoreType.DMA,
        ],
        compiler_params=pltpu.CompilerParams(
            collective_id=0,
            vmem_limit_bytes=56 * 1024 * 1024,
        ),
    )(xb, wq, wo, kt, vt)
